# baseline (speedup 1.0000x reference)
import jax
import jax.numpy as jnp
from jax import lax
from jax.experimental import pallas as pl
from jax.experimental.pallas import tpu as pltpu
from jax.experimental.pallas import tpu_sc as plsc

L = 16
TL = 128


def _make_body(n, d, ninc, nw, chunk):
    rows_per_w = n // nw
    nchunk = rows_per_w // chunk
    tpc = chunk // TL
    gpt_shift = 3
    gpt_mask = (TL // L) - 1
    xrow = ((ninc + 1 + L - 1) // L) * L
    jac_scale = float(ninc) ** d

    def body(y_hbm, grid_hbm, inc_hbm, x_hbm, jac_hbm, grid_v, inc_v, incx_v,
             y_b, x_b, jac_b, sem_in, sem_x, sem_jac):
        cid = lax.axis_index("c")
        sid = lax.axis_index("s")
        wid = sid * 2 + cid
        pltpu.sync_copy(grid_hbm, grid_v)
        pltpu.sync_copy(inc_hbm, inc_v)
        lanes = lax.iota(jnp.int32, L)

        for dd in range(d):
            def ebody(s, carry, dd=dd):
                k = s * L + lanes
                g = plsc.load_gather(grid_v.at[dd], [jnp.minimum(k, ninc)])
                ig = plsc.load_gather(inc_v.at[dd], [jnp.minimum(k, ninc - 1)])
                w = plsc.bitcast(
                    plsc.pack(g, ig, format=plsc.PackFormat.INTERLEAVED),
                    jnp.int32,
                )
                incx_v[dd, pl.ds(s * L, L)] = w
                return carry

            lax.fori_loop(0, xrow // L, ebody, 0)

        base0 = wid * (rows_per_w // TL)

        def tile0(c):
            return pl.multiple_of(base0 + c * tpc, tpc)

        def start_in(c, b):
            pltpu.async_copy(y_hbm.at[pl.ds(tile0(c), tpc)], y_b[b], sem_in[b])

        def wait_in(c, b):
            pltpu.make_async_copy(
                y_hbm.at[pl.ds(tile0(c), tpc)], y_b[b], sem_in[b]
            ).wait()

        def start_out(c, b):
            t0 = tile0(c)
            pltpu.async_copy(x_b[b], x_hbm.at[pl.ds(t0, tpc)], sem_x[b])
            r0 = pl.multiple_of(t0 * TL, chunk)
            pltpu.async_copy(jac_b[b], jac_hbm.at[pl.ds(r0, chunk)], sem_jac[b])

        def wait_out(c, b):
            t0 = tile0(c)
            pltpu.make_async_copy(
                x_b[b], x_hbm.at[pl.ds(t0, tpc)], sem_x[b]
            ).wait()
            r0 = pl.multiple_of(t0 * TL, chunk)
            pltpu.make_async_copy(
                jac_b[b], jac_hbm.at[pl.ds(r0, chunk)], sem_jac[b]
            ).wait()

        def compute(b):
            y_v, x_v, jac_v = y_b[b], x_b[b], jac_b[b]

            @plsc.parallel_loop(0, chunk // L, unroll=4)
            def ibody(i):
                t = i >> gpt_shift
                l0 = (i & gpt_mask) * L
                jac = None
                for dd in range(d):
                    yv = y_v[t, dd, pl.ds(l0, L)]
                    tt = yv * float(ninc)
                    iy = tt.astype(jnp.int32)
                    dy = tt - iy.astype(jnp.float32)
                    w = plsc.load_gather(incx_v.at[dd], [iy])
                    g, ig = plsc.unpack(
                        plsc.bitcast(w, jnp.bfloat16),
                        format=plsc.PackFormat.INTERLEAVED,
                    )
                    x_v[t, dd, pl.ds(l0, L)] = g + ig * dy
                    jac = ig if jac is None else jac * ig
                jac_v[pl.ds(i * L, L)] = jac * jac_scale

        start_in(0, 0)

        def cbody(h, carry):
            for b in range(2):
                c = h * 2 + b
                wait_in(c, b)

                @pl.when(c + 1 < nchunk)
                def _():
                    start_in(c + 1, 1 - b)

                @pl.when(c >= 2)
                def _():
                    wait_out(c - 2, b)

                compute(b)
                start_out(c, b)
            return carry

        lax.fori_loop(0, nchunk // 2, cbody, 0)
        wait_out(nchunk - 2, 0)
        wait_out(nchunk - 1, 1)

    return body


def kernel(y, grid, inc):
    n, d = y.shape
    ninc = inc.shape[1]
    nw = 32
    rows_per_w = n // nw
    chunk = min(2048, rows_per_w)
    nt = n // TL
    xrow = ((ninc + 1 + L - 1) // L) * L
    y3 = y.T.reshape(d, nt, TL).transpose(1, 0, 2)
    mesh = plsc.VectorSubcoreMesh(
        core_axis_name="c", subcore_axis_name="s", num_cores=2, num_subcores=16
    )
    k = pl.kernel(
        _make_body(n, d, ninc, nw, chunk),
        out_type=[
            jax.ShapeDtypeStruct((nt, d, TL), jnp.float32),
            jax.ShapeDtypeStruct((n,), jnp.float32),
        ],
        mesh=mesh,
        compiler_params=pltpu.CompilerParams(
            needs_layout_passes=False, use_tc_tiling_on_sc=False
        ),
        scratch_types=[
            pltpu.VMEM((d, ninc + 1), jnp.float32),
            pltpu.VMEM((d, ninc), jnp.float32),
            pltpu.VMEM((d, xrow), jnp.int32),
            [pltpu.VMEM((chunk // TL, d, TL), jnp.float32) for _ in range(2)],
            [pltpu.VMEM((chunk // TL, d, TL), jnp.float32) for _ in range(2)],
            [pltpu.VMEM((chunk,), jnp.float32) for _ in range(2)],
            [pltpu.SemaphoreType.DMA for _ in range(2)],
            [pltpu.SemaphoreType.DMA for _ in range(2)],
            [pltpu.SemaphoreType.DMA for _ in range(2)],
        ],
    )
    x3, jac = k(y3, grid, inc)
    x = x3.transpose(1, 0, 2).reshape(d, n).T
    return x, jac


# speedup vs baseline: 1.0289x; 1.0289x over previous
import jax
import jax.numpy as jnp
from jax import lax
from jax.experimental import pallas as pl
from jax.experimental.pallas import tpu as pltpu
from jax.experimental.pallas import tpu_sc as plsc

L = 16
TL = 128


def _make_body(n, d, ninc, nw, chunk):
    rows_per_w = n // nw
    nchunk = rows_per_w // chunk
    tpc = chunk // TL
    gpt_shift = 3
    gpt_mask = (TL // L) - 1
    xrow = ((ninc + 1 + L - 1) // L) * L
    jac_scale = float(ninc) ** d

    def body(y_hbm, grid_hbm, inc_hbm, x_hbm, jac_hbm, grid_v, incx_v,
             y_b, x_b, jac_b, sem_in, sem_x, sem_jac):
        cid = lax.axis_index("c")
        sid = lax.axis_index("s")
        wid = sid * 2 + cid
        pltpu.sync_copy(grid_hbm, grid_v)
        lanes = lax.iota(jnp.int32, L)

        for dd in range(d):
            pltpu.sync_copy(inc_hbm.at[dd], incx_v.at[dd, pl.ds(0, ninc)])
        for dd in range(d):
            k = jnp.minimum(xrow - L + lanes, ninc - 1)
            incx_v[dd, pl.ds(xrow - L, L)] = plsc.load_gather(
                incx_v.at[dd], [k]
            )

        base0 = wid * (rows_per_w // TL)

        def tile0(c):
            return pl.multiple_of(base0 + c * tpc, tpc)

        def start_in(c, b):
            pltpu.async_copy(y_hbm.at[pl.ds(tile0(c), tpc)], y_b[b], sem_in[b])

        def wait_in(c, b):
            pltpu.make_async_copy(
                y_hbm.at[pl.ds(tile0(c), tpc)], y_b[b], sem_in[b]
            ).wait()

        def start_out(c, b):
            t0 = tile0(c)
            pltpu.async_copy(x_b[b], x_hbm.at[pl.ds(t0, tpc)], sem_x[b])
            r0 = pl.multiple_of(t0 * TL, chunk)
            pltpu.async_copy(jac_b[b], jac_hbm.at[pl.ds(r0, chunk)], sem_jac[b])

        def wait_out(c, b):
            t0 = tile0(c)
            pltpu.make_async_copy(
                x_b[b], x_hbm.at[pl.ds(t0, tpc)], sem_x[b]
            ).wait()
            r0 = pl.multiple_of(t0 * TL, chunk)
            pltpu.make_async_copy(
                jac_b[b], jac_hbm.at[pl.ds(r0, chunk)], sem_jac[b]
            ).wait()

        def compute(b):
            y_v, x_v, jac_v = y_b[b], x_b[b], jac_b[b]

            @plsc.parallel_loop(0, chunk // L, unroll=8)
            def ibody(i):
                t = i >> gpt_shift
                l0 = (i & gpt_mask) * L
                jac = None
                for dd in range(d):
                    yv = y_v[t, dd, pl.ds(l0, L)]
                    tt = yv * float(ninc)
                    iy = tt.astype(jnp.int32)
                    dy = tt - iy.astype(jnp.float32)
                    g = plsc.load_gather(grid_v.at[dd], [iy])
                    ig = plsc.load_gather(incx_v.at[dd], [iy])
                    x_v[t, dd, pl.ds(l0, L)] = g + ig * dy
                    jac = ig if jac is None else jac * ig
                jac_v[pl.ds(i * L, L)] = jac * jac_scale

        start_in(0, 0)

        def cbody(h, carry):
            for b in range(2):
                c = h * 2 + b
                wait_in(c, b)

                @pl.when(c + 1 < nchunk)
                def _():
                    start_in(c + 1, 1 - b)

                @pl.when(c >= 2)
                def _():
                    wait_out(c - 2, b)

                compute(b)
                start_out(c, b)
            return carry

        lax.fori_loop(0, nchunk // 2, cbody, 0)
        wait_out(nchunk - 2, 0)
        wait_out(nchunk - 1, 1)

    return body


def kernel(y, grid, inc):
    n, d = y.shape
    ninc = inc.shape[1]
    nw = 32
    rows_per_w = n // nw
    chunk = min(2048, rows_per_w)
    nt = n // TL
    xrow = ((ninc + 1 + L - 1) // L) * L
    y3 = y.T.reshape(d, nt, TL).transpose(1, 0, 2)
    mesh = plsc.VectorSubcoreMesh(
        core_axis_name="c", subcore_axis_name="s", num_cores=2, num_subcores=16
    )
    k = pl.kernel(
        _make_body(n, d, ninc, nw, chunk),
        out_type=[
            jax.ShapeDtypeStruct((nt, d, TL), jnp.float32),
            jax.ShapeDtypeStruct((n,), jnp.float32),
        ],
        mesh=mesh,
        compiler_params=pltpu.CompilerParams(
            needs_layout_passes=False, use_tc_tiling_on_sc=False
        ),
        scratch_types=[
            pltpu.VMEM((d, ninc + 1), jnp.float32),
            pltpu.VMEM((d, xrow), jnp.float32),
            [pltpu.VMEM((chunk // TL, d, TL), jnp.float32) for _ in range(2)],
            [pltpu.VMEM((chunk // TL, d, TL), jnp.float32) for _ in range(2)],
            [pltpu.VMEM((chunk,), jnp.float32) for _ in range(2)],
            [pltpu.SemaphoreType.DMA for _ in range(2)],
            [pltpu.SemaphoreType.DMA for _ in range(2)],
            [pltpu.SemaphoreType.DMA for _ in range(2)],
        ],
    )
    x3, jac = k(y3, grid, inc)
    x = x3.transpose(1, 0, 2).reshape(d, n).T
    return x, jac
